# baseline (speedup 1.0000x reference)
import functools

import jax
import jax.numpy as jnp
from jax import lax
from jax.experimental import pallas as pl
from jax.experimental.pallas import tpu as pltpu
from jax.experimental.pallas import tpu_sc as plsc

D_MODEL = 64
SCALE = 8.0
NUM_WORKERS = 32
CHUNK = 800


def kernel(x, table):
    s0, s1 = x.shape
    bsz = s0 * s1
    xf = x.reshape(bsz).astype(jnp.int32)
    b_per_w = bsz // NUM_WORKERS
    n_chunks = b_per_w // CHUNK

    mesh = plsc.VectorSubcoreMesh(core_axis_name="c", subcore_axis_name="s")

    @functools.partial(
        pl.kernel,
        mesh=mesh,
        out_type=jax.ShapeDtypeStruct((bsz, D_MODEL), jnp.float32),
        scratch_types=[
            pltpu.VMEM((CHUNK,), jnp.int32),
            pltpu.VMEM((CHUNK, D_MODEL), jnp.float32),
            pltpu.SemaphoreType.DMA,
        ],
        compiler_params=pltpu.CompilerParams(use_tc_tiling_on_sc=False),
    )
    def emb(x_hbm, table_hbm, out_hbm, idx_v, rows_v, sem):
        wid = lax.axis_index("s") * 2 + lax.axis_index("c")
        base = wid * b_per_w

        def chunk_body(g, carry):
            off = base + g * CHUNK
            pltpu.sync_copy(x_hbm.at[pl.ds(off, CHUNK)], idx_v)
            pltpu.async_copy(table_hbm.at[idx_v], rows_v, sem).wait()

            def row_body(r, c2):
                for c in range(D_MODEL // 16):
                    sl = rows_v[r, pl.ds(c * 16, 16)]
                    rows_v[r, pl.ds(c * 16, 16)] = sl * SCALE
                return c2

            lax.fori_loop(0, CHUNK, row_body, 0)
            pltpu.sync_copy(rows_v, out_hbm.at[pl.ds(off, CHUNK)])
            return carry

        lax.fori_loop(0, n_chunks, chunk_body, 0)

    out = emb(xf, table)
    return out.reshape(s0, s1, D_MODEL)


# speedup vs baseline: 1.1180x; 1.1180x over previous
import functools

import jax
import jax.numpy as jnp
from jax import lax
from jax.experimental import pallas as pl
from jax.experimental.pallas import tpu as pltpu
from jax.experimental.pallas import tpu_sc as plsc

D_MODEL = 64
SCALE = 8.0
NUM_WORKERS = 32
CHUNK = 400
NBUF = 4


def kernel(x, table):
    s0, s1 = x.shape
    bsz = s0 * s1
    xf = x.reshape(bsz).astype(jnp.int32)
    b_per_w = bsz // NUM_WORKERS
    n_chunks = b_per_w // CHUNK
    assert n_chunks % NBUF == 0 and n_chunks >= 2 * NBUF

    mesh = plsc.VectorSubcoreMesh(core_axis_name="c", subcore_axis_name="s")

    @functools.partial(
        pl.kernel,
        mesh=mesh,
        out_type=jax.ShapeDtypeStruct((bsz, D_MODEL), jnp.float32),
        scratch_types=[
            pltpu.VMEM((b_per_w,), jnp.int32),
            [pltpu.VMEM((CHUNK, D_MODEL), jnp.float32) for _ in range(NBUF)],
            [pltpu.SemaphoreType.DMA for _ in range(NBUF)],
            [pltpu.SemaphoreType.DMA for _ in range(NBUF)],
        ],
        compiler_params=pltpu.CompilerParams(use_tc_tiling_on_sc=False),
    )
    def emb(x_hbm, table_hbm, out_hbm, idx_v, rows, gsem, ssem):
        wid = lax.axis_index("s") * 2 + lax.axis_index("c")
        base = wid * b_per_w

        def issue_gather(g, b):
            idx_slice = idx_v.at[pl.ds(g * CHUNK, CHUNK)]
            pltpu.async_copy(table_hbm.at[idx_slice], rows[b], gsem[b])

        def wait_gather(b):
            pltpu.make_async_copy(table_hbm.at[pl.ds(0, CHUNK)], rows[b], gsem[b]).wait()

        def issue_scatter(g, b):
            dst = out_hbm.at[pl.ds(base + g * CHUNK, CHUNK)]
            pltpu.async_copy(rows[b], dst, ssem[b])

        def wait_scatter(b):
            pltpu.make_async_copy(rows[b], out_hbm.at[pl.ds(0, CHUNK)], ssem[b]).wait()

        def turn(g, b, do_scat_wait, do_gat_issue):
            b2 = (b + 2) % NBUF
            if do_scat_wait:
                wait_scatter(b2)
            if do_gat_issue:
                issue_gather(g + 2, b2)
            wait_gather(b)

            @plsc.parallel_loop(0, CHUNK, step=1, unroll=4)
            def _scale(r):
                for c in range(D_MODEL // 16):
                    sl = rows[b][r, pl.ds(c * 16, 16)]
                    rows[b][r, pl.ds(c * 16, 16)] = sl * SCALE

            issue_scatter(g, b)

        pltpu.sync_copy(x_hbm.at[pl.ds(base, b_per_w)], idx_v)
        issue_gather(0, 0)
        issue_gather(1, 1)

        for b in range(NBUF):
            turn(b, b, b >= 2, True)

        def cycle(g4, carry):
            for b in range(NBUF):
                turn(g4 * NBUF + b, b, True, True)
            return carry

        lax.fori_loop(1, n_chunks // NBUF - 1, cycle, 0)

        for b in range(NBUF):
            g = n_chunks - NBUF + b
            turn(g, b, True, b < 2)

        wait_scatter((n_chunks - 2) % NBUF)
        wait_scatter((n_chunks - 1) % NBUF)

    out = emb(xf, table)
    return out.reshape(s0, s1, D_MODEL)
